# baseline (speedup 1.0000x reference)
import functools
import math

import jax
import jax.numpy as jnp
from jax import lax
from jax.experimental import pallas as pl
from jax.experimental.pallas import tpu as pltpu
from jax.experimental.pallas import tpu_sc as plsc

_N = 10000
_E = 320000
_D = 128

_NC = 2
_NS = 16
_NW = _NC * _NS
_EPW = _E // _NW
_CH = 80
_NCHUNK = _EPW // _CH
_NBUF = 5
_NGRP = _NCHUNK // _NBUF

_mesh = plsc.VectorSubcoreMesh(core_axis_name="c", subcore_axis_name="s")


@functools.partial(
    pl.kernel,
    out_type=jax.ShapeDtypeStruct((_NC, 2, _N), jnp.float32),
    mesh=_mesh,
    scratch_types=[
        pltpu.VMEM((_NCHUNK, _CH), jnp.int32),
        pltpu.VMEM((_NCHUNK, _CH), jnp.int32),
        pltpu.VMEM((_CH,), jnp.float32),
        pltpu.VMEM((640,), jnp.float32),
        pltpu.VMEM_SHARED((_N,), jnp.float32),
        pltpu.VMEM_SHARED((_N,), jnp.float32),
        pltpu.SemaphoreType.DMA,
        pltpu.SemaphoreType.DMA,
    ],
)
def _deg_kernel(src_hbm, dst_hbm, out_hbm, sidx, didx, ones, zbuf,
                dego, degi, sem_s, sem_d):
    cid = lax.axis_index("c")
    sid = lax.axis_index("s")
    wid = sid * _NC + cid

    cps = pltpu.async_copy(src_hbm.at[wid], sidx, sem_s)
    cpd = pltpu.async_copy(dst_hbm.at[wid], didx, sem_d)

    for j in range(_CH // 16):
        ones[pl.ds(j * 16, 16)] = jnp.ones((16,), jnp.float32)
    for j in range(640 // 16):
        zbuf[pl.ds(j * 16, 16)] = jnp.zeros((16,), jnp.float32)

    z0 = sid * 624
    pltpu.sync_copy(zbuf, dego.at[pl.ds(z0, 640)])
    pltpu.sync_copy(zbuf, degi.at[pl.ds(z0, 640)])
    cps.wait()
    cpd.wait()
    plsc.subcore_barrier()

    _DEPTH = 8

    def fire(i):
        pltpu.async_copy(ones, dego.at[sidx.at[i]], sem_s, add=True)
        pltpu.async_copy(ones, degi.at[didx.at[i]], sem_d, add=True)

    def drain_one():
        pltpu.make_async_copy(ones, dego.at[sidx.at[0]], sem_s).wait()
        pltpu.make_async_copy(ones, degi.at[didx.at[0]], sem_d).wait()

    def head(i, carry):
        fire(i)
        return carry

    def steady(i, carry):
        fire(i)
        drain_one()
        return carry

    def tail(i, carry):
        drain_one()
        return carry

    lax.fori_loop(0, _DEPTH, head, 0)
    lax.fori_loop(_DEPTH, _NCHUNK, steady, 0)
    lax.fori_loop(0, _DEPTH, tail, 0)
    plsc.subcore_barrier()

    @pl.when(sid == 0)
    def _():
        pltpu.sync_copy(dego, out_hbm.at[cid, 0])
        pltpu.sync_copy(degi, out_hbm.at[cid, 1])


_RB = 4
_IQ = 6
_GRP = math.lcm(_RB, _IQ)


@functools.partial(
    pl.kernel,
    out_type=jax.ShapeDtypeStruct((_NC, _N, _D), jnp.float32),
    mesh=_mesh,
    scratch_types=[
        [pltpu.VMEM((_CH,), jnp.int32)] * _IQ,
        [pltpu.VMEM((_CH,), jnp.int32)] * _IQ,
        [pltpu.VMEM((_CH, _D), jnp.float32)] * _RB,
        pltpu.VMEM_SHARED((_N, _D), jnp.float32),
        [pltpu.SemaphoreType.DMA] * _IQ,
        [pltpu.SemaphoreType.DMA] * _IQ,
        [pltpu.SemaphoreType.DMA] * _RB,
        [pltpu.SemaphoreType.DMA] * _RB,
    ],
)
def _agg_kernel(h_hbm, src_hbm, dst_hbm, out_hbm, sidx, didx, rings, acc,
                isems_s, isems_d, gsems, ssems):
    cid = lax.axis_index("c")
    sid = lax.axis_index("s")
    wid = sid * _NC + cid
    base = wid * _EPW

    def fire_idx(i, q):
        off = base + i * _CH
        pltpu.async_copy(src_hbm.at[pl.ds(off, _CH)], sidx[q], isems_s[q])
        pltpu.async_copy(dst_hbm.at[pl.ds(off, _CH)], didx[q], isems_d[q])

    def wait_idx_s(q):
        pltpu.make_async_copy(
            src_hbm.at[pl.ds(0, _CH)], sidx[q], isems_s[q]).wait()

    def wait_idx_d(q):
        pltpu.make_async_copy(
            dst_hbm.at[pl.ds(0, _CH)], didx[q], isems_d[q]).wait()

    for q in range(_IQ):
        fire_idx(q, q)

    def zrow(r, carry):
        for j in range(_D // 16):
            rings[0][r, pl.ds(j * 16, 16)] = jnp.zeros((16,), jnp.float32)
        return carry

    lax.fori_loop(0, _CH, zrow, 0)

    z0 = sid * 624
    for j in range(640 // _CH):
        pltpu.sync_copy(rings[0], acc.at[pl.ds(z0 + j * _CH, _CH)])
    plsc.subcore_barrier()

    for b in range(_RB):
        wait_idx_s(b)
        pltpu.async_copy(h_hbm.at[sidx[b]], rings[b], gsems[b])

    def step(i, b, q, refill_idx, regather):
        pltpu.make_async_copy(h_hbm.at[sidx[q]], rings[b], gsems[b]).wait()
        wait_idx_d(q)
        pltpu.async_copy(rings[b], acc.at[didx[q]], ssems[b], add=True)
        pltpu.make_async_copy(rings[b], acc.at[didx[q]], ssems[b]).wait()
        if refill_idx:
            fire_idx(i + _IQ, q)
        if regather:
            qn = (q + _RB) % _IQ
            wait_idx_s(qn)
            pltpu.async_copy(h_hbm.at[sidx[qn]], rings[b], gsems[b])

    def group(g, carry):
        for j in range(_GRP):
            i = g * _GRP + j
            step(i, j % _RB, j % _IQ, True, True)
        return carry

    _NSTEADY = (_NCHUNK - _IQ) // _GRP
    lax.fori_loop(0, _NSTEADY, group, 0)
    for i in range(_NSTEADY * _GRP, _NCHUNK):
        step(i, i % _RB, i % _IQ, i + _IQ < _NCHUNK, i + _RB < _NCHUNK)
    plsc.subcore_barrier()

    pltpu.sync_copy(acc.at[pl.ds(z0, 640)],
                    out_hbm.at[cid, pl.ds(z0, 640)])


def _norm(deg):
    return lax.rsqrt(jnp.maximum(deg, 1.0))


def _mm0_body(x_ref, w_ref, o_ref):
    o_ref[...] = jnp.dot(x_ref[...], w_ref[...],
                         preferred_element_type=jnp.float32)


def _l1_body(y_ref, dp_ref, o_ref):
    ns = _norm(dp_ref[0, 0] + dp_ref[1, 0])
    o_ref[...] = y_ref[...] * ns


def _l2_body(ap_ref, dp_ref, b_ref, w_ref, o_ref):
    agg = ap_ref[0] + ap_ref[1]
    nd = _norm(dp_ref[0, 1] + dp_ref[1, 1])
    ns = _norm(dp_ref[0, 0] + dp_ref[1, 0])
    h = jnp.maximum(agg * nd + b_ref[...], 0.0)
    o_ref[...] = jnp.dot(h * ns, w_ref[...],
                         preferred_element_type=jnp.float32)


def _l3_body(ap_ref, dp_ref, b_ref, o_ref):
    nd = _norm(dp_ref[0, 1] + dp_ref[1, 1])
    o_ref[...] = (ap_ref[0] + ap_ref[1]) * nd + b_ref[...]


_mm0 = pl.pallas_call(
    _mm0_body, out_shape=jax.ShapeDtypeStruct((_N, _D), jnp.float32))
_l1 = pl.pallas_call(
    _l1_body, out_shape=jax.ShapeDtypeStruct((_N, _D), jnp.float32))
_l2 = pl.pallas_call(
    _l2_body, out_shape=jax.ShapeDtypeStruct((_N, _D), jnp.float32))
_l3 = pl.pallas_call(
    _l3_body, out_shape=jax.ShapeDtypeStruct((_N, _D), jnp.float32))


def kernel(x, edge_index, W0, b0, W1, b1):
    src = edge_index[0]
    dst = edge_index[1]
    src3 = src.reshape(_NW, _NCHUNK, _CH)
    dst3 = dst.reshape(_NW, _NCHUNK, _CH)
    degp = _deg_kernel(src3, dst3)
    y0 = _mm0(x, W0)
    degp = degp.reshape(_NC, 2, _N, 1)
    h0 = _l1(y0, degp)
    aggp0 = _agg_kernel(h0, src, dst)
    h1 = _l2(aggp0, degp, b0.reshape(1, _D), W1)
    aggp1 = _agg_kernel(h1, src, dst)
    return _l3(aggp1, degp, b1.reshape(1, _D))


# speedup vs baseline: 1.0048x; 1.0048x over previous
import functools
import math

import jax
import jax.numpy as jnp
from jax import lax
from jax.experimental import pallas as pl
from jax.experimental.pallas import tpu as pltpu
from jax.experimental.pallas import tpu_sc as plsc

_N = 10000
_E = 320000
_D = 128

_NC = 2
_NS = 16
_NW = _NC * _NS
_EPW = _E // _NW
_CH = 80
_NCHUNK = _EPW // _CH
_NBUF = 5
_NGRP = _NCHUNK // _NBUF

_mesh = plsc.VectorSubcoreMesh(core_axis_name="c", subcore_axis_name="s")


@functools.partial(
    pl.kernel,
    out_type=jax.ShapeDtypeStruct((_NC, 2, _N), jnp.float32),
    mesh=_mesh,
    scratch_types=[
        pltpu.VMEM((_NCHUNK, _CH), jnp.int32),
        pltpu.VMEM((_NCHUNK, _CH), jnp.int32),
        pltpu.VMEM((_CH,), jnp.float32),
        pltpu.VMEM((640,), jnp.float32),
        pltpu.VMEM_SHARED((_N,), jnp.float32),
        pltpu.VMEM_SHARED((_N,), jnp.float32),
        pltpu.SemaphoreType.DMA,
        pltpu.SemaphoreType.DMA,
    ],
)
def _deg_kernel(src_hbm, dst_hbm, out_hbm, sidx, didx, ones, zbuf,
                dego, degi, sem_s, sem_d):
    cid = lax.axis_index("c")
    sid = lax.axis_index("s")
    wid = sid * _NC + cid

    cps = pltpu.async_copy(src_hbm.at[wid], sidx, sem_s)
    cpd = pltpu.async_copy(dst_hbm.at[wid], didx, sem_d)

    for j in range(_CH // 16):
        ones[pl.ds(j * 16, 16)] = jnp.ones((16,), jnp.float32)
    for j in range(640 // 16):
        zbuf[pl.ds(j * 16, 16)] = jnp.zeros((16,), jnp.float32)

    z0 = sid * 624
    pltpu.sync_copy(zbuf, dego.at[pl.ds(z0, 640)])
    pltpu.sync_copy(zbuf, degi.at[pl.ds(z0, 640)])
    cps.wait()
    cpd.wait()
    plsc.subcore_barrier()

    _DEPTH = 8

    def fire(i):
        pltpu.async_copy(ones, dego.at[sidx.at[i]], sem_s, add=True)
        pltpu.async_copy(ones, degi.at[didx.at[i]], sem_d, add=True)

    def drain_one():
        pltpu.make_async_copy(ones, dego.at[sidx.at[0]], sem_s).wait()
        pltpu.make_async_copy(ones, degi.at[didx.at[0]], sem_d).wait()

    def head(i, carry):
        fire(i)
        return carry

    def steady(i, carry):
        fire(i)
        drain_one()
        return carry

    def tail(i, carry):
        drain_one()
        return carry

    lax.fori_loop(0, _DEPTH, head, 0)
    lax.fori_loop(_DEPTH, _NCHUNK, steady, 0)
    lax.fori_loop(0, _DEPTH, tail, 0)
    plsc.subcore_barrier()

    @pl.when(sid == 0)
    def _():
        pltpu.sync_copy(dego, out_hbm.at[cid, 0])
        pltpu.sync_copy(degi, out_hbm.at[cid, 1])


_RB = 4
_SL = 8
_NSLAB = _NCHUNK // _SL
_TAIL = _NCHUNK - _NSLAB * _SL


@functools.partial(
    pl.kernel,
    out_type=jax.ShapeDtypeStruct((_NC, _N, _D), jnp.float32),
    mesh=_mesh,
    scratch_types=[
        [pltpu.VMEM((_SL, _CH), jnp.int32)] * 2,
        [pltpu.VMEM((_SL, _CH), jnp.int32)] * 2,
        [pltpu.VMEM((_CH, _D), jnp.float32)] * _RB,
        pltpu.VMEM_SHARED((_N, _D), jnp.float32),
        [pltpu.SemaphoreType.DMA] * 2,
        [pltpu.SemaphoreType.DMA] * _RB,
        [pltpu.SemaphoreType.DMA] * _RB,
    ],
)
def _agg_kernel(h_hbm, src_hbm, dst_hbm, out_hbm, sslab, dslab, rings, acc,
                isems, gsems, ssems):
    cid = lax.axis_index("c")
    sid = lax.axis_index("s")
    wid = sid * _NC + cid

    def fire_slab(sbase, slot):
        pltpu.async_copy(src_hbm.at[wid, pl.ds(sbase, _SL)],
                         sslab[slot], isems[slot])
        pltpu.async_copy(dst_hbm.at[wid, pl.ds(sbase, _SL)],
                         dslab[slot], isems[slot])

    def wait_slab(slot):
        pltpu.make_async_copy(src_hbm.at[0, pl.ds(0, _SL)],
                              sslab[slot], isems[slot]).wait()
        pltpu.make_async_copy(dst_hbm.at[0, pl.ds(0, _SL)],
                              dslab[slot], isems[slot]).wait()

    def fire_slab_tail(slot):
        pltpu.async_copy(src_hbm.at[wid, pl.ds(_NSLAB * _SL, _TAIL)],
                         sslab[slot].at[pl.ds(0, _TAIL)], isems[slot])
        pltpu.async_copy(dst_hbm.at[wid, pl.ds(_NSLAB * _SL, _TAIL)],
                         dslab[slot].at[pl.ds(0, _TAIL)], isems[slot])

    def wait_slab_tail(slot):
        pltpu.make_async_copy(src_hbm.at[0, pl.ds(0, _TAIL)],
                              sslab[slot].at[pl.ds(0, _TAIL)],
                              isems[slot]).wait()
        pltpu.make_async_copy(dst_hbm.at[0, pl.ds(0, _TAIL)],
                              dslab[slot].at[pl.ds(0, _TAIL)],
                              isems[slot]).wait()

    fire_slab(0, 0)
    fire_slab(_SL, 1)

    def zrow(r, carry):
        for j in range(_D // 16):
            rings[0][r, pl.ds(j * 16, 16)] = jnp.zeros((16,), jnp.float32)
        return carry

    lax.fori_loop(0, _CH, zrow, 0)

    z0 = sid * 624
    for j in range(640 // _CH):
        pltpu.sync_copy(rings[0], acc.at[pl.ds(z0 + j * _CH, _CH)])
    plsc.subcore_barrier()

    wait_slab(0)
    for j in range(_RB):
        pltpu.async_copy(h_hbm.at[sslab[0].at[j]], rings[j], gsems[j])

    def step(j, slot, nslot, last_j):
        b = j % _RB
        pltpu.make_async_copy(h_hbm.at[sslab[slot].at[j]],
                              rings[b], gsems[b]).wait()
        pltpu.async_copy(rings[b], acc.at[dslab[slot].at[j]], ssems[b],
                         add=True)
        pltpu.make_async_copy(rings[b], acc.at[dslab[slot].at[j]],
                              ssems[b]).wait()
        jn = j + _RB
        if jn <= last_j:
            if jn < _SL:
                pltpu.async_copy(h_hbm.at[sslab[slot].at[jn]],
                                 rings[b], gsems[b])
            else:
                pltpu.async_copy(h_hbm.at[sslab[nslot].at[jn - _SL]],
                                 rings[b], gsems[b])

    def slab_body(slot, nchunks, next_wait):
        nslot = 1 - slot
        last_j = (_SL + _RB - 1) if next_wait else (nchunks - 1)
        for j in range(nchunks):
            if next_wait and j == _SL - _RB:
                if next_wait == "tail":
                    wait_slab_tail(nslot)
                else:
                    wait_slab(nslot)
            step(j, slot, nslot, last_j)

    slab_body(0, _SL, "full")
    fire_slab(2 * _SL, 0)

    def pair(g, carry):
        sb = (2 * g + 1) * _SL
        slab_body(1, _SL, "full")
        fire_slab(sb + 2 * _SL, 1)
        slab_body(0, _SL, "full")
        fire_slab(sb + 3 * _SL, 0)
        return carry

    lax.fori_loop(0, 5, pair, 0)

    slab_body(1, _SL, "full")
    fire_slab(13 * _SL, 1)
    slab_body(0, _SL, "full")
    fire_slab(14 * _SL, 0)
    slab_body(1, _SL, "full")
    fire_slab_tail(1)
    slab_body(0, _SL, "tail")
    slab_body(1, _TAIL, None)
    plsc.subcore_barrier()

    pltpu.sync_copy(acc.at[pl.ds(z0, 640)],
                    out_hbm.at[cid, pl.ds(z0, 640)])


def _norm(deg):
    return lax.rsqrt(jnp.maximum(deg, 1.0))


def _l1_body(x_ref, dp_ref, w_ref, o_ref):
    ns = _norm(dp_ref[0, 0] + dp_ref[1, 0])
    o_ref[...] = jnp.dot(x_ref[...] * ns, w_ref[...],
                         preferred_element_type=jnp.float32)


def _l2_body(ap_ref, dp_ref, b_ref, w_ref, o_ref):
    agg = ap_ref[0] + ap_ref[1]
    nd = _norm(dp_ref[0, 1] + dp_ref[1, 1])
    ns = _norm(dp_ref[0, 0] + dp_ref[1, 0])
    h = jnp.maximum(agg * nd + b_ref[...], 0.0)
    o_ref[...] = jnp.dot(h * ns, w_ref[...],
                         preferred_element_type=jnp.float32)


def _l3_body(ap_ref, dp_ref, b_ref, o_ref):
    nd = _norm(dp_ref[0, 1] + dp_ref[1, 1])
    o_ref[...] = (ap_ref[0] + ap_ref[1]) * nd + b_ref[...]


_l1 = pl.pallas_call(
    _l1_body, out_shape=jax.ShapeDtypeStruct((_N, _D), jnp.float32))
_l2 = pl.pallas_call(
    _l2_body, out_shape=jax.ShapeDtypeStruct((_N, _D), jnp.float32))
_l3 = pl.pallas_call(
    _l3_body, out_shape=jax.ShapeDtypeStruct((_N, _D), jnp.float32))


def kernel(x, edge_index, W0, b0, W1, b1):
    src = edge_index[0]
    dst = edge_index[1]
    src3 = src.reshape(_NW, _NCHUNK, _CH)
    dst3 = dst.reshape(_NW, _NCHUNK, _CH)
    degp = _deg_kernel(src3, dst3)
    degp = degp.reshape(_NC, 2, _N, 1)
    h0 = _l1(x, degp, W0)
    aggp0 = _agg_kernel(h0, src3, dst3)
    h1 = _l2(aggp0, degp, b0.reshape(1, _D), W1)
    aggp1 = _agg_kernel(h1, src3, dst3)
    return _l3(aggp1, degp, b1.reshape(1, _D))


# speedup vs baseline: 1.0152x; 1.0103x over previous
import functools

import jax
import jax.numpy as jnp
from jax import lax
from jax.experimental import pallas as pl
from jax.experimental.pallas import tpu as pltpu
from jax.experimental.pallas import tpu_sc as plsc

_N = 10000
_E = 320000
_D = 128

_NC = 2
_NS = 16
_NW = _NC * _NS
_EPW = _E // _NW
_CH = 80
_NCHUNK = _EPW // _CH

_mesh = plsc.VectorSubcoreMesh(core_axis_name="c", subcore_axis_name="s")


@functools.partial(
    pl.kernel,
    out_type=jax.ShapeDtypeStruct((_NC, 2, _N), jnp.float32),
    mesh=_mesh,
    scratch_types=[
        pltpu.VMEM((_NCHUNK, _CH), jnp.int32),
        pltpu.VMEM((_NCHUNK, _CH), jnp.int32),
        pltpu.VMEM((_CH,), jnp.float32),
        pltpu.VMEM((640,), jnp.float32),
        pltpu.VMEM_SHARED((_N,), jnp.float32),
        pltpu.VMEM_SHARED((_N,), jnp.float32),
        pltpu.SemaphoreType.DMA,
        pltpu.SemaphoreType.DMA,
    ],
)
def _deg_kernel(src_hbm, dst_hbm, out_hbm, sidx, didx, ones, zbuf,
                dego, degi, sem_s, sem_d):
    cid = lax.axis_index("c")
    sid = lax.axis_index("s")
    wid = sid * _NC + cid

    cps = pltpu.async_copy(src_hbm.at[wid], sidx, sem_s)
    cpd = pltpu.async_copy(dst_hbm.at[wid], didx, sem_d)

    for j in range(_CH // 16):
        ones[pl.ds(j * 16, 16)] = jnp.ones((16,), jnp.float32)
    for j in range(640 // 16):
        zbuf[pl.ds(j * 16, 16)] = jnp.zeros((16,), jnp.float32)

    z0 = sid * 624
    pltpu.sync_copy(zbuf, dego.at[pl.ds(z0, 640)])
    pltpu.sync_copy(zbuf, degi.at[pl.ds(z0, 640)])
    cps.wait()
    cpd.wait()
    plsc.subcore_barrier()

    _DEPTH = 8

    def fire(i):
        pltpu.async_copy(ones, dego.at[sidx.at[i]], sem_s, add=True)
        pltpu.async_copy(ones, degi.at[didx.at[i]], sem_d, add=True)

    def drain_one():
        pltpu.make_async_copy(ones, dego.at[sidx.at[0]], sem_s).wait()
        pltpu.make_async_copy(ones, degi.at[didx.at[0]], sem_d).wait()

    def head(i, carry):
        fire(i)
        return carry

    def steady(i, carry):
        fire(i)
        drain_one()
        return carry

    def tail(i, carry):
        drain_one()
        return carry

    lax.fori_loop(0, _DEPTH, head, 0)
    lax.fori_loop(_DEPTH, _NCHUNK, steady, 0)
    lax.fori_loop(0, _DEPTH, tail, 0)
    plsc.subcore_barrier()

    @pl.when(sid == 0)
    def _():
        pltpu.sync_copy(dego, out_hbm.at[cid, 0])
        pltpu.sync_copy(degi, out_hbm.at[cid, 1])


_RB = 4
_SL = 8
_NSLAB = _NCHUNK // _SL
_TAIL = _NCHUNK - _NSLAB * _SL


@functools.partial(
    pl.kernel,
    out_type=jax.ShapeDtypeStruct((_NC, _N, _D), jnp.float32),
    mesh=_mesh,
    scratch_types=[
        [pltpu.VMEM((_SL, _CH), jnp.int32)] * 2,
        [pltpu.VMEM((_SL, _CH), jnp.int32)] * 2,
        [pltpu.VMEM((_CH, _D), jnp.float32)] * _RB,
        pltpu.VMEM_SHARED((_N, _D), jnp.float32),
        [pltpu.SemaphoreType.DMA] * 2,
        [pltpu.SemaphoreType.DMA] * _RB,
        [pltpu.SemaphoreType.DMA] * _RB,
    ],
)
def _agg_kernel(h_hbm, src_hbm, dst_hbm, out_hbm, sslab, dslab, rings, acc,
                isems, gsems, ssems):
    cid = lax.axis_index("c")
    sid = lax.axis_index("s")
    wid = sid * _NC + cid

    def fire_slab(sbase, slot):
        pltpu.async_copy(src_hbm.at[wid, pl.ds(sbase, _SL)],
                         sslab[slot], isems[slot])
        pltpu.async_copy(dst_hbm.at[wid, pl.ds(sbase, _SL)],
                         dslab[slot], isems[slot])

    def wait_slab(slot):
        pltpu.make_async_copy(src_hbm.at[0, pl.ds(0, _SL)],
                              sslab[slot], isems[slot]).wait()
        pltpu.make_async_copy(dst_hbm.at[0, pl.ds(0, _SL)],
                              dslab[slot], isems[slot]).wait()

    def fire_slab_tail(slot):
        pltpu.async_copy(src_hbm.at[wid, pl.ds(_NSLAB * _SL, _TAIL)],
                         sslab[slot].at[pl.ds(0, _TAIL)], isems[slot])
        pltpu.async_copy(dst_hbm.at[wid, pl.ds(_NSLAB * _SL, _TAIL)],
                         dslab[slot].at[pl.ds(0, _TAIL)], isems[slot])

    def wait_slab_tail(slot):
        pltpu.make_async_copy(src_hbm.at[0, pl.ds(0, _TAIL)],
                              sslab[slot].at[pl.ds(0, _TAIL)],
                              isems[slot]).wait()
        pltpu.make_async_copy(dst_hbm.at[0, pl.ds(0, _TAIL)],
                              dslab[slot].at[pl.ds(0, _TAIL)],
                              isems[slot]).wait()

    fire_slab(0, 0)
    fire_slab(_SL, 1)

    def zrow(r, carry):
        for j in range(_D // 16):
            rings[0][r, pl.ds(j * 16, 16)] = jnp.zeros((16,), jnp.float32)
        return carry

    lax.fori_loop(0, _CH, zrow, 0)

    z0 = sid * 624
    for j in range(640 // _CH):
        pltpu.sync_copy(rings[0], acc.at[pl.ds(z0 + j * _CH, _CH)])
    plsc.subcore_barrier()

    wait_slab(0)
    for j in range(_RB):
        pltpu.async_copy(h_hbm.at[sslab[0].at[j]], rings[j], gsems[j])

    def step(j, slot, nslot, last_j):
        b = j % _RB
        pltpu.make_async_copy(h_hbm.at[sslab[slot].at[j]],
                              rings[b], gsems[b]).wait()
        pltpu.async_copy(rings[b], acc.at[dslab[slot].at[j]], ssems[b],
                         add=True)
        pltpu.make_async_copy(rings[b], acc.at[dslab[slot].at[j]],
                              ssems[b]).wait()
        jn = j + _RB
        if jn <= last_j:
            if jn < _SL:
                pltpu.async_copy(h_hbm.at[sslab[slot].at[jn]],
                                 rings[b], gsems[b])
            else:
                pltpu.async_copy(h_hbm.at[sslab[nslot].at[jn - _SL]],
                                 rings[b], gsems[b])

    def slab_body(slot, nchunks, next_wait):
        nslot = 1 - slot
        last_j = (_SL + _RB - 1) if next_wait else (nchunks - 1)
        for j in range(nchunks):
            if next_wait and j == _SL - _RB:
                if next_wait == "tail":
                    wait_slab_tail(nslot)
                else:
                    wait_slab(nslot)
            step(j, slot, nslot, last_j)

    slab_body(0, _SL, "full")
    fire_slab(2 * _SL, 0)

    def pair(g, carry):
        sb = (2 * g + 1) * _SL
        slab_body(1, _SL, "full")
        fire_slab(sb + 2 * _SL, 1)
        slab_body(0, _SL, "full")
        fire_slab(sb + 3 * _SL, 0)
        return carry

    lax.fori_loop(0, 5, pair, 0)

    slab_body(1, _SL, "full")
    fire_slab(13 * _SL, 1)
    slab_body(0, _SL, "full")
    fire_slab(14 * _SL, 0)
    slab_body(1, _SL, "full")
    fire_slab_tail(1)
    slab_body(0, _SL, "tail")
    slab_body(1, _TAIL, None)
    plsc.subcore_barrier()

    pltpu.sync_copy(acc.at[pl.ds(z0, 640)],
                    out_hbm.at[cid, pl.ds(z0, 640)])


def _norm(deg):
    return lax.rsqrt(jnp.maximum(deg, 1.0))


_RBLK = 2000


def _l1_body(x_ref, dp_ref, w_ref, o_ref):
    ns = _norm(dp_ref[0, 0] + dp_ref[1, 0])
    o_ref[...] = jnp.dot(x_ref[...] * ns, w_ref[...],
                         preferred_element_type=jnp.float32)


def _l2_body(ap_ref, dp_ref, b_ref, w_ref, o_ref):
    agg = ap_ref[0] + ap_ref[1]
    nd = _norm(dp_ref[0, 1] + dp_ref[1, 1])
    ns = _norm(dp_ref[0, 0] + dp_ref[1, 0])
    h = jnp.maximum(agg * nd + b_ref[...], 0.0)
    o_ref[...] = jnp.dot(h * ns, w_ref[...],
                         preferred_element_type=jnp.float32)


def _l3_body(ap_ref, dp_ref, b_ref, o_ref):
    nd = _norm(dp_ref[0, 1] + dp_ref[1, 1])
    o_ref[...] = (ap_ref[0] + ap_ref[1]) * nd + b_ref[...]


_out_spec = pl.BlockSpec((_RBLK, _D), lambda i: (i, 0))
_dp_spec = pl.BlockSpec((2, 2, _RBLK, 1), lambda i: (0, 0, i, 0))
_ap_spec = pl.BlockSpec((2, _RBLK, _D), lambda i: (0, i, 0))
_w_spec = pl.BlockSpec((_D, _D), lambda i: (0, 0))
_b_spec = pl.BlockSpec((1, _D), lambda i: (0, 0))

_l1 = pl.pallas_call(
    _l1_body,
    grid=(_N // _RBLK,),
    in_specs=[_out_spec, _dp_spec, _w_spec],
    out_specs=_out_spec,
    out_shape=jax.ShapeDtypeStruct((_N, _D), jnp.float32))
_l2 = pl.pallas_call(
    _l2_body,
    grid=(_N // _RBLK,),
    in_specs=[_ap_spec, _dp_spec, _b_spec, _w_spec],
    out_specs=_out_spec,
    out_shape=jax.ShapeDtypeStruct((_N, _D), jnp.float32))
_l3 = pl.pallas_call(
    _l3_body,
    grid=(_N // _RBLK,),
    in_specs=[_ap_spec, _dp_spec, _b_spec],
    out_specs=_out_spec,
    out_shape=jax.ShapeDtypeStruct((_N, _D), jnp.float32))


def kernel(x, edge_index, W0, b0, W1, b1):
    src = edge_index[0]
    dst = edge_index[1]
    src3 = src.reshape(_NW, _NCHUNK, _CH)
    dst3 = dst.reshape(_NW, _NCHUNK, _CH)
    degp = _deg_kernel(src3, dst3)
    degp = degp.reshape(_NC, 2, _N, 1)
    h0 = _l1(x, degp, W0)
    aggp0 = _agg_kernel(h0, src3, dst3)
    h1 = _l2(aggp0, degp, b0.reshape(1, _D), W1)
    aggp1 = _agg_kernel(h1, src3, dst3)
    return _l3(aggp1, degp, b1.reshape(1, _D))
